# baseline (speedup 1.0000x reference)
import functools

import jax
import jax.numpy as jnp
import numpy as np
from jax import lax
from jax.experimental import pallas as pl
from jax.experimental.pallas import tpu as pltpu
from jax.experimental.pallas import tpu_sc as plsc

_GENETIC_CODE = {'TTT': 'F', 'TTC': 'F', 'TTA': 'L', 'TTG': 'L', 'CTT': 'L', 'CTC': 'L', 'CTA': 'L', 'CTG': 'L', 'ATT': 'I', 'ATC': 'I', 'ATA': 'I', 'ATG': 'M', 'GTT': 'V', 'GTC': 'V', 'GTA': 'V', 'GTG': 'V', 'TCT': 'S', 'TCC': 'S', 'TCA': 'S', 'TCG': 'S', 'AGT': 'S', 'AGC': 'S', 'CCT': 'P', 'CCC': 'P', 'CCA': 'P', 'CCG': 'P', 'ACT': 'T', 'ACC': 'T', 'ACA': 'T', 'ACG': 'T', 'GCT': 'A', 'GCC': 'A', 'GCA': 'A', 'GCG': 'A', 'TAT': 'Y', 'TAC': 'Y', 'TAA': '*', 'TAG': '*', 'TGA': '*', 'CAT': 'H', 'CAC': 'H', 'CAA': 'Q', 'CAG': 'Q', 'AAT': 'N', 'AAC': 'N', 'AAA': 'K', 'AAG': 'K', 'GAT': 'D', 'GAC': 'D', 'GAA': 'E', 'GAG': 'E', 'TGT': 'C', 'TGC': 'C', 'TGG': 'W', 'CGT': 'R', 'CGC': 'R', 'CGA': 'R', 'CGG': 'R', 'AGA': 'R', 'AGG': 'R', 'GGT': 'G', 'GGC': 'G', 'GGA': 'G', 'GGG': 'G'}
_AA_ORDER = 'ACDEFGHIKLMNPQRSTVWY*'
_NUC = {'T': 0, 'C': 1, 'A': 2, 'G': 3}


def _codon_aa_onehot() -> np.ndarray:
    aa_idx = {aa: i for i, aa in enumerate(_AA_ORDER)}
    m = np.zeros((64, 24), dtype=np.float32)
    for codon, aa in _GENETIC_CODE.items():
        c = _NUC[codon[0]] * 16 + _NUC[codon[1]] * 4 + _NUC[codon[2]]
        m[c, aa_idx[aa]] = 1.0
    return m


_ONEHOT = _codon_aa_onehot()


def _erf_f32(x):
    a1, a2, a3, a4, a5 = 0.254829592, -0.284496736, 1.421413741, -1.453152027, 1.061405429
    s = jnp.sign(x)
    ax = jnp.abs(x)
    t = 1.0 / (1.0 + 0.3275911 * ax)
    poly = ((((a5 * t + a4) * t + a3) * t + a2) * t + a1) * t
    return s * (1.0 - poly * jnp.exp(-ax * ax))


def _table_body(onehot_ref, aa_ref, cdev_ref, pe_ref, wob_ref, w1_ref, b1_ref,
                w2_ref, b2_ref, g_ref, beta_ref, out_ref):
    emb64 = jnp.dot(onehot_ref[...], aa_ref[...],
                    preferred_element_type=jnp.float32) + cdev_ref[...]
    rows = []
    for p in range(3):
        pe = pe_ref[...][p:p + 1, :]
        wv = wob_ref[...][p:p + 1, :]
        rows.append((emb64 + pe) * wv)
    emb = jnp.concatenate(rows, axis=0)
    h = jnp.dot(emb, w1_ref[...], preferred_element_type=jnp.float32) + b1_ref[...]
    h = 0.5 * h * (1.0 + _erf_f32(h * 0.7071067811865476))
    h = jnp.dot(h, w2_ref[...], preferred_element_type=jnp.float32) + b2_ref[...]
    mu = jnp.mean(h, axis=1, keepdims=True)
    xc = h - mu
    var = jnp.mean(xc * xc, axis=1, keepdims=True)
    out_ref[...] = xc * lax.rsqrt(var + 1e-5) * g_ref[...] + beta_ref[...]


def _compute_table(aa_emb, codon_dev, pos_enc, wobble_weights, W1, b1, W2, b2,
                   ln_g, ln_b):
    aa_pad = jnp.zeros((24, 64), jnp.float32).at[:21].set(aa_emb)
    pe_pad = jnp.zeros((8, 64), jnp.float32).at[:3].set(pos_enc)
    wob_pad = jnp.zeros((8, 64), jnp.float32).at[:3].set(
        jnp.broadcast_to(wobble_weights[:, None], (3, 64)))
    return pl.pallas_call(
        _table_body,
        out_shape=jax.ShapeDtypeStruct((192, 64), jnp.float32),
    )(jnp.asarray(_ONEHOT), aa_pad, codon_dev, pe_pad, wob_pad,
      W1, b1.reshape(1, -1), W2, b2.reshape(1, -1),
      ln_g.reshape(1, -1), ln_b.reshape(1, -1))


_L = 200
_LB = _L - 128
_IDX_BLK = 512


def _idx_body(cod_ref, outa_ref, outb_ref):
    c = lax.broadcasted_iota(jnp.int32, (_IDX_BLK, _L), 1)
    idx = cod_ref[...] + (c % 3) * 64
    outa_ref[...] = idx[:, :128]
    outb_ref[:, :_LB] = idx[:, 128:]


def _compute_idx(codons):
    B = codons.shape[0]
    return pl.pallas_call(
        _idx_body,
        grid=(B // _IDX_BLK,),
        in_specs=[pl.BlockSpec((_IDX_BLK, _L), lambda i: (i, 0))],
        out_specs=[pl.BlockSpec((_IDX_BLK, 128), lambda i: (i, 0)),
                   pl.BlockSpec((_IDX_BLK, 128), lambda i: (i, 0))],
        out_shape=[jax.ShapeDtypeStruct((B, 128), jnp.int32),
                   jax.ShapeDtypeStruct((B, 128), jnp.int32)],
    )(codons)


_CR = 2
_NBUF = 3


def _make_gather(B: int):
    info = plsc.get_sparse_core_info()
    nc, ns = info.num_cores, info.num_subcores
    nw = nc * ns
    assert B % (nw * _CR) == 0
    seqs_per_w = B // nw
    n_chunks = seqs_per_w // _CR
    mesh = plsc.VectorSubcoreMesh(core_axis_name="c", subcore_axis_name="s")

    scratch = ([pltpu.VMEM((_CR * 128,), jnp.int32)] * (2 * _NBUF)
               + [pltpu.VMEM((_CR, _L, 64), jnp.float32)] * _NBUF
               + [pltpu.SemaphoreType.DMA] * (3 * _NBUF))

    @functools.partial(
        pl.kernel, mesh=mesh,
        compiler_params=pltpu.CompilerParams(use_tc_tiling_on_sc=False),
        out_type=jax.ShapeDtypeStruct((B, _L, 64), jnp.float32),
        scratch_types=scratch,
    )
    def gather_k(idxa_hbm, idxb_hbm, table_hbm, out_hbm, *scr):
        idxa_v = scr[:_NBUF]
        idxb_v = scr[_NBUF:2 * _NBUF]
        rows_v = scr[2 * _NBUF:3 * _NBUF]
        sem_i = scr[3 * _NBUF:4 * _NBUF]
        sem_g = scr[4 * _NBUF:5 * _NBUF]
        sem_s = scr[5 * _NBUF:]
        wid = lax.axis_index("s") * nc + lax.axis_index("c")
        seq0 = wid * seqs_per_w

        def fire_idx(i):
            b = i % _NBUF
            off = (seq0 + i * _CR) * 128
            ha = pltpu.async_copy(
                idxa_hbm.at[pl.ds(off, _CR * 128)], idxa_v[b], sem_i[b])
            hb = pltpu.async_copy(
                idxb_hbm.at[pl.ds(off, _CR * 128)], idxb_v[b], sem_i[b])
            return (ha, hb)

        h_idx = [None] * n_chunks
        h_s = [None] * n_chunks
        for i in range(min(_NBUF, n_chunks)):
            h_idx[i] = fire_idx(i)
        for i in range(n_chunks):
            b = i % _NBUF
            h_idx[i][0].wait()
            h_idx[i][1].wait()
            if i >= _NBUF:
                h_s[i - _NBUF].wait()
            hg = []
            for j in range(_CR):
                hg.append(pltpu.async_copy(
                    table_hbm.at[idxa_v[b].at[pl.ds(j * 128, 128)]],
                    rows_v[b].at[j, pl.ds(0, 128)], sem_g[b]))
                hg.append(pltpu.async_copy(
                    table_hbm.at[idxb_v[b].at[pl.ds(j * 128, _LB)]],
                    rows_v[b].at[j, pl.ds(128, _LB)], sem_g[b]))
            for h in hg:
                h.wait()
            if i + _NBUF < n_chunks:
                h_idx[i + _NBUF] = fire_idx(i + _NBUF)
            h_s[i] = pltpu.async_copy(
                rows_v[b], out_hbm.at[pl.ds(seq0 + i * _CR, _CR)],
                sem_s[b])
        for i in range(max(0, n_chunks - _NBUF), n_chunks):
            h_s[i].wait()

    return gather_k


def kernel(codons, aa_emb, codon_dev, pos_enc, wobble_weights, W1, b1, W2, b2,
           ln_g, ln_b):
    B, L = codons.shape
    table = _compute_table(aa_emb, codon_dev, pos_enc, wobble_weights,
                           W1, b1, W2, b2, ln_g, ln_b)
    idxa, idxb = _compute_idx(codons)
    h = B // 2
    gather = _make_gather(h)
    o1 = gather(idxa[:h].reshape(h * 128), idxb[:h].reshape(h * 128), table)
    o2 = gather(idxa[h:].reshape(h * 128), idxb[h:].reshape(h * 128), table)
    return jnp.concatenate([o1, o2], axis=0)


# speedup vs baseline: 1.2022x; 1.2022x over previous
import functools

import jax
import jax.numpy as jnp
import numpy as np
from jax import lax
from jax.experimental import pallas as pl
from jax.experimental.pallas import tpu as pltpu
from jax.experimental.pallas import tpu_sc as plsc

_GENETIC_CODE = {'TTT': 'F', 'TTC': 'F', 'TTA': 'L', 'TTG': 'L', 'CTT': 'L', 'CTC': 'L', 'CTA': 'L', 'CTG': 'L', 'ATT': 'I', 'ATC': 'I', 'ATA': 'I', 'ATG': 'M', 'GTT': 'V', 'GTC': 'V', 'GTA': 'V', 'GTG': 'V', 'TCT': 'S', 'TCC': 'S', 'TCA': 'S', 'TCG': 'S', 'AGT': 'S', 'AGC': 'S', 'CCT': 'P', 'CCC': 'P', 'CCA': 'P', 'CCG': 'P', 'ACT': 'T', 'ACC': 'T', 'ACA': 'T', 'ACG': 'T', 'GCT': 'A', 'GCC': 'A', 'GCA': 'A', 'GCG': 'A', 'TAT': 'Y', 'TAC': 'Y', 'TAA': '*', 'TAG': '*', 'TGA': '*', 'CAT': 'H', 'CAC': 'H', 'CAA': 'Q', 'CAG': 'Q', 'AAT': 'N', 'AAC': 'N', 'AAA': 'K', 'AAG': 'K', 'GAT': 'D', 'GAC': 'D', 'GAA': 'E', 'GAG': 'E', 'TGT': 'C', 'TGC': 'C', 'TGG': 'W', 'CGT': 'R', 'CGC': 'R', 'CGA': 'R', 'CGG': 'R', 'AGA': 'R', 'AGG': 'R', 'GGT': 'G', 'GGC': 'G', 'GGA': 'G', 'GGG': 'G'}
_AA_ORDER = 'ACDEFGHIKLMNPQRSTVWY*'
_NUC = {'T': 0, 'C': 1, 'A': 2, 'G': 3}


def _codon_aa_onehot() -> np.ndarray:
    aa_idx = {aa: i for i, aa in enumerate(_AA_ORDER)}
    m = np.zeros((64, 24), dtype=np.float32)
    for codon, aa in _GENETIC_CODE.items():
        c = _NUC[codon[0]] * 16 + _NUC[codon[1]] * 4 + _NUC[codon[2]]
        m[c, aa_idx[aa]] = 1.0
    return m


_ONEHOT = _codon_aa_onehot()


def _erf_f32(x):
    a1, a2, a3, a4, a5 = 0.254829592, -0.284496736, 1.421413741, -1.453152027, 1.061405429
    s = jnp.sign(x)
    ax = jnp.abs(x)
    t = 1.0 / (1.0 + 0.3275911 * ax)
    poly = ((((a5 * t + a4) * t + a3) * t + a2) * t + a1) * t
    return s * (1.0 - poly * jnp.exp(-ax * ax))


def _table_body(onehot_ref, aa_ref, cdev_ref, pe_ref, wob_ref, w1_ref, b1_ref,
                w2_ref, b2_ref, g_ref, beta_ref, out_ref):
    emb64 = jnp.dot(onehot_ref[...], aa_ref[...],
                    preferred_element_type=jnp.float32) + cdev_ref[...]
    rows = []
    for p in range(3):
        pe = pe_ref[...][p:p + 1, :]
        wv = wob_ref[...][p:p + 1, :]
        rows.append((emb64 + pe) * wv)
    emb = jnp.concatenate(rows, axis=0)
    h = jnp.dot(emb, w1_ref[...], preferred_element_type=jnp.float32) + b1_ref[...]
    h = 0.5 * h * (1.0 + _erf_f32(h * 0.7071067811865476))
    h = jnp.dot(h, w2_ref[...], preferred_element_type=jnp.float32) + b2_ref[...]
    mu = jnp.mean(h, axis=1, keepdims=True)
    xc = h - mu
    var = jnp.mean(xc * xc, axis=1, keepdims=True)
    out_ref[...] = xc * lax.rsqrt(var + 1e-5) * g_ref[...] + beta_ref[...]


def _compute_table(aa_emb, codon_dev, pos_enc, wobble_weights, W1, b1, W2, b2,
                   ln_g, ln_b):
    aa_pad = jnp.zeros((24, 64), jnp.float32).at[:21].set(aa_emb)
    pe_pad = jnp.zeros((8, 64), jnp.float32).at[:3].set(pos_enc)
    wob_pad = jnp.zeros((8, 64), jnp.float32).at[:3].set(
        jnp.broadcast_to(wobble_weights[:, None], (3, 64)))
    return pl.pallas_call(
        _table_body,
        out_shape=jax.ShapeDtypeStruct((192, 64), jnp.float32),
    )(jnp.asarray(_ONEHOT), aa_pad, codon_dev, pe_pad, wob_pad,
      W1, b1.reshape(1, -1), W2, b2.reshape(1, -1),
      ln_g.reshape(1, -1), ln_b.reshape(1, -1))


_L = 200
_LB = _L - 128
_IDX_BLK = 512


def _idx_body(cod_ref, outa_ref, outb_ref):
    c = lax.broadcasted_iota(jnp.int32, (_IDX_BLK, _L), 1)
    idx = cod_ref[...] + (c % 3) * 64
    outa_ref[...] = idx[:, :128]
    outb_ref[:, :_LB] = idx[:, 128:]


def _compute_idx(codons):
    B = codons.shape[0]
    return pl.pallas_call(
        _idx_body,
        grid=(B // _IDX_BLK,),
        in_specs=[pl.BlockSpec((_IDX_BLK, _L), lambda i: (i, 0))],
        out_specs=[pl.BlockSpec((_IDX_BLK, 128), lambda i: (i, 0)),
                   pl.BlockSpec((_IDX_BLK, 128), lambda i: (i, 0))],
        out_shape=[jax.ShapeDtypeStruct((B, 128), jnp.int32),
                   jax.ShapeDtypeStruct((B, 128), jnp.int32)],
    )(codons)


_CR = 2
_NBUF = 3


def _make_gather(B: int):
    info = plsc.get_sparse_core_info()
    nc, ns = info.num_cores, info.num_subcores
    nw = nc * ns
    assert B % (nw * _CR) == 0
    seqs_per_w = B // nw
    n_chunks = seqs_per_w // _CR
    mesh = plsc.VectorSubcoreMesh(core_axis_name="c", subcore_axis_name="s")

    scratch = ([pltpu.VMEM((_CR * 128,), jnp.int32)] * (2 * _NBUF)
               + [pltpu.VMEM((_CR, _L, 64), jnp.float32)] * _NBUF
               + [pltpu.SemaphoreType.DMA] * (3 * _NBUF))

    @functools.partial(
        pl.kernel, mesh=mesh,
        compiler_params=pltpu.CompilerParams(use_tc_tiling_on_sc=False),
        out_type=jax.ShapeDtypeStruct((B, _L, 64), jnp.float32),
        scratch_types=scratch,
    )
    def gather_k(idxa_hbm, idxb_hbm, table_hbm, out_hbm, *scr):
        idxa_v = scr[:_NBUF]
        idxb_v = scr[_NBUF:2 * _NBUF]
        rows_v = scr[2 * _NBUF:3 * _NBUF]
        sem_i = scr[3 * _NBUF:4 * _NBUF]
        sem_g = scr[4 * _NBUF:5 * _NBUF]
        sem_s = scr[5 * _NBUF:]
        wid = lax.axis_index("s") * nc + lax.axis_index("c")
        seq0 = wid * seqs_per_w

        def fire_idx(i):
            b = i % _NBUF
            off = (seq0 + i * _CR) * 128
            ha = pltpu.async_copy(
                idxa_hbm.at[pl.ds(off, _CR * 128)], idxa_v[b], sem_i[b])
            hb = pltpu.async_copy(
                idxb_hbm.at[pl.ds(off, _CR * 128)], idxb_v[b], sem_i[b])
            return (ha, hb)

        h_idx = [None] * n_chunks
        h_s = [None] * n_chunks
        for i in range(min(_NBUF, n_chunks)):
            h_idx[i] = fire_idx(i)
        for i in range(n_chunks):
            b = i % _NBUF
            h_idx[i][0].wait()
            h_idx[i][1].wait()
            if i >= _NBUF:
                h_s[i - _NBUF].wait()
            hg = []
            for j in range(_CR):
                hg.append(pltpu.async_copy(
                    table_hbm.at[idxa_v[b].at[pl.ds(j * 128, 128)]],
                    rows_v[b].at[j, pl.ds(0, 128)], sem_g[b]))
                hg.append(pltpu.async_copy(
                    table_hbm.at[idxb_v[b].at[pl.ds(j * 128, _LB)]],
                    rows_v[b].at[j, pl.ds(128, _LB)], sem_g[b]))
            for h in hg:
                h.wait()
            if i + _NBUF < n_chunks:
                h_idx[i + _NBUF] = fire_idx(i + _NBUF)
            h_s[i] = pltpu.async_copy(
                rows_v[b], out_hbm.at[pl.ds(seq0 + i * _CR, _CR)],
                sem_s[b])
        for i in range(max(0, n_chunks - _NBUF), n_chunks):
            h_s[i].wait()

    return gather_k


def kernel(codons, aa_emb, codon_dev, pos_enc, wobble_weights, W1, b1, W2, b2,
           ln_g, ln_b):
    B, L = codons.shape
    table = _compute_table(aa_emb, codon_dev, pos_enc, wobble_weights,
                           W1, b1, W2, b2, ln_g, ln_b)
    idxa, idxb = _compute_idx(codons)
    return _make_gather(B)(idxa.reshape(B * 128), idxb.reshape(B * 128), table)
